# baseline (device time: 256228 ns/iter reference)
import functools

import jax
import jax.numpy as jnp
from jax import lax
from jax.experimental import pallas as pl
from jax.experimental.pallas import tpu as pltpu

X, Y, Z = 2, 2, 4
M = 2048
HR = 1024
S4 = 512
F4 = 256
C4 = 64
KS = 1024

RSYS, RSZS, AGZS, AGYS, AGX1S, AGX2S = 0, 1, 4, 7, 8, 9
NSEM = 20

MESH = pl.DeviceIdType.MESH


def kernel(dy, W):
    xi = lax.axis_index("x")
    zi = lax.axis_index("z")
    h0 = xi * HR
    ca = zi * KS
    cb = (Z + zi) * KS
    dy_a = lax.dynamic_slice(dy, (h0, ca), (HR, KS))
    dy_b = lax.dynamic_slice(dy, (h0, cb), (HR, KS))
    w_a = lax.dynamic_slice(W, (0, ca), (M, KS))
    w_b = lax.dynamic_slice(W, (0, cb), (M, KS))

    def body(dya_ref, dyb_ref, wa_ref, wb_ref, out_ref, CY, CZ,
             ssem, rsem, cred0, cred1):
        x = lax.axis_index("x")
        y = lax.axis_index("y")
        z = lax.axis_index("z")
        xp = (1 - x, y, z)
        yp = (x, 1 - y, z)
        zl = (x, y, (z - 1) % Z)
        zr = (x, y, (z + 1) % Z)
        h0 = x * HR
        creds = [cred0, cred1]

        def mm(lr, nrows):
            out_ref[pl.ds(h0 + lr, nrows), :] = lax.dot_general(
                dya_ref[pl.ds(lr, nrows), :], wa_ref[...],
                dimension_numbers=(((1,), (1,)), ((), ())),
                preferred_element_type=jnp.float32,
            ) + lax.dot_general(
                dyb_ref[pl.ds(lr, nrows), :], wb_ref[...],
                dimension_numbers=(((1,), (1,)), ((), ())),
                preferred_element_type=jnp.float32,
            )

        def copy(src, dst, sem_i, dev):
            return pltpu.make_async_remote_copy(
                src_ref=src, dst_ref=dst,
                send_sem=ssem.at[sem_i], recv_sem=rsem.at[sem_i],
                device_id=dev, device_id_type=MESH,
            )

        def add(r0, nrows, buf):
            out_ref[pl.ds(r0, nrows), :] = (
                out_ref[pl.ds(r0, nrows), :] + buf
            )

        class Chain:
            def __init__(self, i, lbase, d):
                self.i, self.d, self.sb = i, d, 10 * i
                self.cred = creds[i]
                self.f_my_l = lbase + y * F4
                self.f_oth_l = lbase + (1 - y) * F4
                self.f_my = h0 + self.f_my_l
                self.f_oth = h0 + self.f_oth_l
                self.snbr = zr if d == 1 else zl
                self.rnbr = zl if d == 1 else zr
                self.r = None
                self.rx1 = None
                self.rx2 = None

            def rs_send(self, s):
                return (z - self.d * s) % Z

            def rs_recv(self, s):
                return (z - self.d * (s + 1)) % Z

            def ag_send(self, s):
                return (z + self.d * (1 - s)) % Z

            def p1_start(self):
                self.r = copy(out_ref.at[pl.ds(self.f_oth, F4), :],
                              CY.at[self.i], self.sb + RSYS, yp)
                self.r.start()

            def p1_fin(self):
                self.r.wait()
                add(self.f_my, F4, CY[self.i])

            def rs_start(self, s):
                if s == 2:
                    pl.semaphore_wait(self.cred, 1)
                rows = self.f_my + self.rs_send(s) * C4
                self.r = copy(out_ref.at[pl.ds(rows, C4), :],
                              CZ.at[self.i, s % 2], self.sb + RSZS + s,
                              self.snbr)
                self.r.start()

            def rs_fin(self, s):
                self.r.wait()
                add(self.f_my + self.rs_recv(s) * C4, C4, CZ[self.i, s % 2])
                if s == 0:
                    pl.semaphore_signal(self.cred, inc=1,
                                        device_id=self.rnbr,
                                        device_id_type=MESH)

            def ag_start(self, s):
                rows = self.f_my + self.ag_send(s) * C4
                self.r = copy(out_ref.at[pl.ds(rows, C4), :],
                              out_ref.at[pl.ds(rows, C4), :],
                              self.sb + AGZS + s, self.snbr)
                self.r.start()

            def ag_fin(self, s):
                self.r.wait()

            def agy_start(self):
                self.r = copy(out_ref.at[pl.ds(self.f_my, F4), :],
                              out_ref.at[pl.ds(self.f_my, F4), :],
                              self.sb + AGYS, yp)
                self.r.start()

            def agy_fin(self):
                self.r.wait()

            def agx1_start(self):
                self.rx1 = copy(out_ref.at[pl.ds(self.f_my, F4), :],
                                out_ref.at[pl.ds(self.f_my, F4), :],
                                self.sb + AGX1S, xp)
                self.rx1.start()

            def agx1_fin(self):
                self.rx1.wait()

            def agx2_start(self):
                self.rx2 = copy(out_ref.at[pl.ds(self.f_oth, F4), :],
                                out_ref.at[pl.ds(self.f_oth, F4), :],
                                self.sb + AGX2S, xp)
                self.rx2.start()

            def agx2_fin(self):
                self.rx2.wait()

        chains = [Chain(0, 0, 1), Chain(1, S4, -1)]
        c0, c1 = chains

        barrier = pltpu.get_barrier_semaphore()
        for nbr in (xp, yp, zl, zr):
            pl.semaphore_signal(barrier, inc=1, device_id=nbr,
                                device_id_type=MESH)
        pl.semaphore_wait(barrier, 4)

        mm(c0.f_oth_l, F4)
        c0.p1_start()
        mm(c1.f_oth_l, F4)
        c1.p1_start()
        mm(c0.f_my_l, F4)
        mm(c1.f_my_l, F4)

        for c in chains:
            c.p1_fin()
            c.rs_start(0)
        for s in range(3):
            for c in chains:
                c.rs_fin(s)
                if s < 2:
                    c.rs_start(s + 1)
                else:
                    c.ag_start(0)
        for s in range(3):
            for c in chains:
                c.ag_fin(s)
                if s < 2:
                    c.ag_start(s + 1)
                else:
                    c.agx1_start()
                    c.agy_start()
        for c in chains:
            c.agy_fin()
            c.agx2_start()
        for c in chains:
            c.agx1_fin()
            c.agx2_fin()

        @functools.partial(pl.run_scoped, sem2=pltpu.SemaphoreType.REGULAR)
        def _(sem2):
            for nbr in (xp, yp, zl, zr):
                pl.semaphore_signal(sem2, inc=1, device_id=nbr,
                                    device_id_type=MESH)
            pl.semaphore_wait(sem2, 4)

    return pl.pallas_call(
        body,
        out_shape=jax.ShapeDtypeStruct((M, M), jnp.float32),
        in_specs=[
            pl.BlockSpec(memory_space=pltpu.VMEM),
            pl.BlockSpec(memory_space=pltpu.VMEM),
            pl.BlockSpec(memory_space=pltpu.VMEM),
            pl.BlockSpec(memory_space=pltpu.VMEM),
        ],
        out_specs=pl.BlockSpec(memory_space=pltpu.VMEM),
        scratch_shapes=[
            pltpu.VMEM((2, F4, M), jnp.float32),
            pltpu.VMEM((2, 2, C4, M), jnp.float32),
            pltpu.SemaphoreType.DMA((NSEM,)),
            pltpu.SemaphoreType.DMA((NSEM,)),
            pltpu.SemaphoreType.REGULAR,
            pltpu.SemaphoreType.REGULAR,
        ],
        compiler_params=pltpu.CompilerParams(
            collective_id=0,
            vmem_limit_bytes=60 * 1024 * 1024,
        ),
    )(dy_a, dy_b, w_a, w_b)


# device time: 231595 ns/iter; 1.1064x vs baseline; 1.1064x over previous
import functools

import jax
import jax.numpy as jnp
from jax import lax
from jax.experimental import pallas as pl
from jax.experimental.pallas import tpu as pltpu

X, Y, Z = 2, 2, 4
M = 2048
HR = 1024
S4 = 512
F4 = 256
C4 = 64
KS = 1024

RSYS, RSZS, AGZS, AGYS, AGX1S, AGX2S = 0, 1, 4, 7, 8, 9
NSEM = 20

MESH = pl.DeviceIdType.MESH


def kernel(dy, W):
    def body(dy_ref, w_ref, out_ref, dya_ref, dyb_ref, wa_ref, wb_ref,
             CY, CZ, lsem, ssem, rsem, cred0, cred1):
        x = lax.axis_index("x")
        y = lax.axis_index("y")
        z = lax.axis_index("z")
        xp = (1 - x, y, z)
        yp = (x, 1 - y, z)
        zl = (x, y, (z - 1) % Z)
        zr = (x, y, (z + 1) % Z)
        h0 = x * HR
        ca = z * KS
        cb = (Z + z) * KS
        creds = [cred0, cred1]

        cps = [
            pltpu.make_async_copy(
                dy_ref.at[pl.ds(h0, HR), pl.ds(ca, KS)], dya_ref, lsem.at[0]),
            pltpu.make_async_copy(
                dy_ref.at[pl.ds(h0, HR), pl.ds(cb, KS)], dyb_ref, lsem.at[1]),
            pltpu.make_async_copy(
                w_ref.at[:, pl.ds(ca, KS)], wa_ref, lsem.at[2]),
            pltpu.make_async_copy(
                w_ref.at[:, pl.ds(cb, KS)], wb_ref, lsem.at[3]),
        ]
        for cp in cps:
            cp.start()

        def mm(lr, nrows):
            out_ref[pl.ds(h0 + lr, nrows), :] = lax.dot_general(
                dya_ref[pl.ds(lr, nrows), :], wa_ref[...],
                dimension_numbers=(((1,), (1,)), ((), ())),
                preferred_element_type=jnp.float32,
            ) + lax.dot_general(
                dyb_ref[pl.ds(lr, nrows), :], wb_ref[...],
                dimension_numbers=(((1,), (1,)), ((), ())),
                preferred_element_type=jnp.float32,
            )

        def copy(src, dst, sem_i, dev):
            return pltpu.make_async_remote_copy(
                src_ref=src, dst_ref=dst,
                send_sem=ssem.at[sem_i], recv_sem=rsem.at[sem_i],
                device_id=dev, device_id_type=MESH,
            )

        def add(r0, nrows, buf):
            out_ref[pl.ds(r0, nrows), :] = (
                out_ref[pl.ds(r0, nrows), :] + buf
            )

        class Chain:
            def __init__(self, i, lbase, d):
                self.i, self.d, self.sb = i, d, 10 * i
                self.cred = creds[i]
                self.f_my_l = lbase + y * F4
                self.f_oth_l = lbase + (1 - y) * F4
                self.f_my = h0 + self.f_my_l
                self.f_oth = h0 + self.f_oth_l
                self.snbr = zr if d == 1 else zl
                self.rnbr = zl if d == 1 else zr
                self.r = None
                self.rx1 = None
                self.rx2 = None

            def rs_send(self, s):
                return (z - self.d * s) % Z

            def rs_recv(self, s):
                return (z - self.d * (s + 1)) % Z

            def ag_send(self, s):
                return (z + self.d * (1 - s)) % Z

            def p1_start(self):
                self.r = copy(out_ref.at[pl.ds(self.f_oth, F4), :],
                              CY.at[self.i], self.sb + RSYS, yp)
                self.r.start()

            def p1_fin(self):
                self.r.wait()
                add(self.f_my, F4, CY[self.i])

            def rs_start(self, s):
                if s == 2:
                    pl.semaphore_wait(self.cred, 1)
                rows = self.f_my + self.rs_send(s) * C4
                self.r = copy(out_ref.at[pl.ds(rows, C4), :],
                              CZ.at[self.i, s % 2], self.sb + RSZS + s,
                              self.snbr)
                self.r.start()

            def rs_fin(self, s):
                self.r.wait()
                add(self.f_my + self.rs_recv(s) * C4, C4, CZ[self.i, s % 2])
                if s == 0:
                    pl.semaphore_signal(self.cred, inc=1,
                                        device_id=self.rnbr,
                                        device_id_type=MESH)

            def ag_start(self, s):
                rows = self.f_my + self.ag_send(s) * C4
                self.r = copy(out_ref.at[pl.ds(rows, C4), :],
                              out_ref.at[pl.ds(rows, C4), :],
                              self.sb + AGZS + s, self.snbr)
                self.r.start()

            def ag_fin(self, s):
                self.r.wait()

            def agy_start(self):
                self.r = copy(out_ref.at[pl.ds(self.f_my, F4), :],
                              out_ref.at[pl.ds(self.f_my, F4), :],
                              self.sb + AGYS, yp)
                self.r.start()

            def agy_fin(self):
                self.r.wait()

            def agx1_start(self):
                self.rx1 = copy(out_ref.at[pl.ds(self.f_my, F4), :],
                                out_ref.at[pl.ds(self.f_my, F4), :],
                                self.sb + AGX1S, xp)
                self.rx1.start()

            def agx1_fin(self):
                self.rx1.wait()

            def agx2_start(self):
                self.rx2 = copy(out_ref.at[pl.ds(self.f_oth, F4), :],
                                out_ref.at[pl.ds(self.f_oth, F4), :],
                                self.sb + AGX2S, xp)
                self.rx2.start()

            def agx2_fin(self):
                self.rx2.wait()

        chains = [Chain(0, 0, 1), Chain(1, S4, -1)]
        c0, c1 = chains

        barrier = pltpu.get_barrier_semaphore()
        for nbr in (xp, yp, zl, zr):
            pl.semaphore_signal(barrier, inc=1, device_id=nbr,
                                device_id_type=MESH)
        pl.semaphore_wait(barrier, 4)
        for cp in cps:
            cp.wait()

        mm(c0.f_oth_l, F4)
        c0.p1_start()
        mm(c1.f_oth_l, F4)
        c1.p1_start()
        mm(c0.f_my_l, F4)
        mm(c1.f_my_l, F4)

        for c in chains:
            c.p1_fin()
            c.rs_start(0)
        for s in range(3):
            for c in chains:
                c.rs_fin(s)
                if s < 2:
                    c.rs_start(s + 1)
                else:
                    c.ag_start(0)
        for s in range(3):
            for c in chains:
                c.ag_fin(s)
                if s < 2:
                    c.ag_start(s + 1)
                else:
                    c.agx1_start()
                    c.agy_start()
        for c in chains:
            c.agy_fin()
            c.agx2_start()
        for c in chains:
            c.agx1_fin()
            c.agx2_fin()

        @functools.partial(pl.run_scoped, sem2=pltpu.SemaphoreType.REGULAR)
        def _(sem2):
            for nbr in (xp, yp, zl, zr):
                pl.semaphore_signal(sem2, inc=1, device_id=nbr,
                                    device_id_type=MESH)
            pl.semaphore_wait(sem2, 4)

    return pl.pallas_call(
        body,
        out_shape=jax.ShapeDtypeStruct((M, M), jnp.float32),
        in_specs=[
            pl.BlockSpec(memory_space=pltpu.MemorySpace.HBM),
            pl.BlockSpec(memory_space=pltpu.MemorySpace.HBM),
        ],
        out_specs=pl.BlockSpec(memory_space=pltpu.VMEM),
        scratch_shapes=[
            pltpu.VMEM((HR, KS), jnp.float32),
            pltpu.VMEM((HR, KS), jnp.float32),
            pltpu.VMEM((M, KS), jnp.float32),
            pltpu.VMEM((M, KS), jnp.float32),
            pltpu.VMEM((2, F4, M), jnp.float32),
            pltpu.VMEM((2, 2, C4, M), jnp.float32),
            pltpu.SemaphoreType.DMA((4,)),
            pltpu.SemaphoreType.DMA((NSEM,)),
            pltpu.SemaphoreType.DMA((NSEM,)),
            pltpu.SemaphoreType.REGULAR,
            pltpu.SemaphoreType.REGULAR,
        ],
        compiler_params=pltpu.CompilerParams(
            collective_id=0,
            vmem_limit_bytes=60 * 1024 * 1024,
        ),
    )(dy, W)
